# baseline (speedup 1.0000x reference)
import functools

import jax
import jax.numpy as jnp
from jax import lax
from jax.experimental import pallas as pl
from jax.experimental.pallas import tpu as pltpu
from jax.experimental.pallas import tpu_sc as plsc

BATCH = 16384
DEPTH = 1000
NUM_CORES = 2
NUM_SUBCORES = 16
NUM_WORKERS = NUM_CORES * NUM_SUBCORES
PER_W = BATCH // NUM_WORKERS
ZROWS = 16
ZELEMS = ZROWS * DEPTH
NZDMA = PER_W // ZROWS
LANES = 16
IDXW = 128
NSCAT = PER_W // IDXW


def _one_hot_body(idx_hbm, out_hbm, idx_v, zero_v, ones_v, pos_v, semz, sems):
    wid = lax.axis_index("s") * NUM_CORES + lax.axis_index("c")
    base = wid * PER_W
    obase = base * DEPTH

    pltpu.sync_copy(idx_hbm.at[pl.ds(base * 1, PER_W)], idx_v)

    zeros16 = jnp.zeros((LANES,), jnp.float32)

    def _zero(i, _):
        off = pl.multiple_of(i * (4 * LANES), 4 * LANES)
        for u in range(4):
            zero_v[pl.ds(off + u * LANES, LANES)] = zeros16
        return _

    lax.fori_loop(0, ZELEMS // (4 * LANES), _zero, None)

    zhandles = [
        pltpu.async_copy(
            zero_v, out_hbm.at[pl.ds(obase + k * ZELEMS, ZELEMS)], semz
        )
        for k in range(NZDMA)
    ]

    ones16 = jnp.full((LANES,), 1.0, jnp.float32)
    iota16 = lax.iota(jnp.int32, LANES)
    for m in range(IDXW // LANES):
        ones_v[pl.ds(m * LANES, LANES)] = ones16
    for g in range(PER_W // LANES):
        idx16 = idx_v[pl.ds(g * LANES, LANES)]
        pos16 = (base + g * LANES + iota16) * DEPTH + idx16
        pos_v[g // (IDXW // LANES), pl.ds((g % (IDXW // LANES)) * LANES, LANES)] = pos16

    for h in zhandles:
        h.wait()

    shandles = [
        pltpu.async_copy(ones_v, out_hbm.at[pos_v.at[j]], sems)
        for j in range(NSCAT)
    ]
    for h in shandles:
        h.wait()


@functools.partial(jax.jit, static_argnames=())
def _one_hot_sc(idx):
    mesh = plsc.VectorSubcoreMesh(core_axis_name="c", subcore_axis_name="s")
    k = functools.partial(
        pl.kernel,
        mesh=mesh,
        out_type=jax.ShapeDtypeStruct((BATCH * DEPTH,), jnp.float32),
        scratch_types=[
            pltpu.VMEM((PER_W,), jnp.int32),
            pltpu.VMEM((ZELEMS,), jnp.float32),
            pltpu.VMEM((IDXW,), jnp.float32),
            pltpu.VMEM((NSCAT, IDXW), jnp.int32),
            pltpu.SemaphoreType.DMA,
            pltpu.SemaphoreType.DMA,
        ],
        compiler_params=pltpu.CompilerParams(needs_layout_passes=False),
    )(_one_hot_body)
    return k(idx)


def kernel(X_in, ones):
    del ones
    flat = _one_hot_sc(X_in.astype(jnp.int32))
    return flat.reshape(BATCH, DEPTH)


# speedup vs baseline: 1.7748x; 1.7748x over previous
import functools

import jax
import jax.numpy as jnp
from jax import lax
from jax.experimental import pallas as pl
from jax.experimental.pallas import tpu as pltpu
from jax.experimental.pallas import tpu_sc as plsc

BATCH = 16384
DEPTH = 1000
NUM_CORES = 2
NUM_SUBCORES = 16
NUM_WORKERS = NUM_CORES * NUM_SUBCORES
PER_W = BATCH // NUM_WORKERS
R = 16
CHUNKS = PER_W // R
LANES = 16
ROW_OFFS = tuple(range(0, DEPTH - LANES, LANES)) + (DEPTH - LANES,)


def _one_hot_body(idx_hbm, out_hbm, idx_v, buf0, buf1, sem0, sem1):
    wid = lax.axis_index("s") * NUM_CORES + lax.axis_index("c")
    base = wid * PER_W

    pltpu.sync_copy(idx_hbm.at[pl.ds(base * 1, PER_W)], idx_v)

    zeros16 = jnp.zeros((LANES,), jnp.float32)

    def _zero(r, _):
        for off in ROW_OFFS:
            buf0[r, pl.ds(off, LANES)] = zeros16
            buf1[r, pl.ds(off, LANES)] = zeros16
        return _

    lax.fori_loop(0, R, _zero, None)

    bufs = (buf0, buf1)
    sems = (sem0, sem1)
    ones16 = jnp.full((LANES,), 1.0, jnp.float32)
    iota16 = lax.iota(jnp.int32, LANES)
    groups = R // LANES

    def scatter_chunk(buf, c, vals):
        for g in range(groups):
            idx16 = idx_v[pl.ds(c * R + g * LANES, LANES)]
            rows16 = g * LANES + iota16
            plsc.store_scatter(buf, [rows16, idx16], vals)

    handles = [None, None]
    for c in range(CHUNKS):
        b = c % 2
        if handles[b] is not None:
            handles[b].wait()
            scatter_chunk(bufs[b], c - 2, zeros16)
        scatter_chunk(bufs[b], c, ones16)
        handles[b] = pltpu.async_copy(
            bufs[b],
            out_hbm.at[pl.ds(base + c * R, R)],
            sems[b],
        )
    handles[0].wait()
    handles[1].wait()


@functools.partial(jax.jit, static_argnames=())
def _one_hot_sc(idx):
    mesh = plsc.VectorSubcoreMesh(core_axis_name="c", subcore_axis_name="s")
    k = functools.partial(
        pl.kernel,
        mesh=mesh,
        out_type=jax.ShapeDtypeStruct((BATCH, DEPTH), jnp.float32),
        scratch_types=[
            pltpu.VMEM((PER_W,), jnp.int32),
            pltpu.VMEM((R, DEPTH), jnp.float32),
            pltpu.VMEM((R, DEPTH), jnp.float32),
            pltpu.SemaphoreType.DMA,
            pltpu.SemaphoreType.DMA,
        ],
        compiler_params=pltpu.CompilerParams(needs_layout_passes=False),
    )(_one_hot_body)
    return k(idx)


def kernel(X_in, ones):
    del ones
    return _one_hot_sc(X_in.astype(jnp.int32))
